# baseline (speedup 1.0000x reference)
import jax
import jax.numpy as jnp
from jax.experimental import pallas as pl

WINDOW = 512
HEAD_DIM = 64
NUM_HEADS = 12
BQ = 256
KB = WINDOW + BQ


def _attn_kernel(q_ref, k_ref, v_ref, o_ref):
    i = pl.program_id(1)
    scale = 1.0 / (HEAD_DIM ** 0.5)
    kstart = jnp.maximum(i - 2, 0) * BQ
    q_idx = i * BQ + jax.lax.broadcasted_iota(jnp.int32, (BQ, KB), 0)
    kv_idx = kstart + jax.lax.broadcasted_iota(jnp.int32, (BQ, KB), 1)
    diff = q_idx - kv_idx
    mask = (diff >= 0) & (diff <= WINDOW)
    bias = jnp.where(mask, jnp.float32(0), jnp.float32(float("-inf")))
    for hh in range(NUM_HEADS):
        c0 = hh * HEAD_DIM
        qh = q_ref[0, :, c0:c0 + HEAD_DIM] * scale
        kh = k_ref[0, pl.ds(kstart, KB), c0:c0 + HEAD_DIM]
        vh = v_ref[0, pl.ds(kstart, KB), c0:c0 + HEAD_DIM]
        s = jax.lax.dot_general(
            qh, kh, (((1,), (1,)), ((), ())),
            preferred_element_type=jnp.float32) + bias
        m = jnp.max(s, axis=-1, keepdims=True)
        p = jnp.exp(s - m)
        denom = jnp.sum(p, axis=-1, keepdims=True)
        oh = jax.lax.dot_general(
            p, vh, (((1,), (0,)), ((), ())),
            preferred_element_type=jnp.float32)
        o_ref[0, :, c0:c0 + HEAD_DIM] = oh * (1.0 / denom)


def kernel(qkv):
    b, l, three, h, e = qkv.shape
    ch = h * e
    nq = l // BQ

    qp = qkv[:, :, 0].reshape(b, l, ch)
    kp = qkv[:, :, 1].reshape(b, l, ch)
    vp = qkv[:, :, 2].reshape(b, l, ch)

    out = pl.pallas_call(
        _attn_kernel,
        grid=(b, nq),
        in_specs=[
            pl.BlockSpec((1, BQ, ch), lambda ib, i: (ib, i, 0)),
            pl.BlockSpec((1, l, ch), lambda ib, i: (ib, 0, 0)),
            pl.BlockSpec((1, l, ch), lambda ib, i: (ib, 0, 0)),
        ],
        out_specs=pl.BlockSpec((1, BQ, ch), lambda ib, i: (ib, i, 0)),
        out_shape=jax.ShapeDtypeStruct((b, l, ch), jnp.float32),
    )(qp, kp, vp)

    return out.reshape(b, l, h, e)


# speedup vs baseline: 1.0172x; 1.0172x over previous
import jax
import jax.numpy as jnp
from jax.experimental import pallas as pl
from jax.experimental.pallas import tpu as pltpu

WINDOW = 512
HEAD_DIM = 64
NUM_HEADS = 12
CH = NUM_HEADS * HEAD_DIM
BQ = 256
KB = WINDOW + BQ


def _attn_kernel(x_ref, o_ref, q_s, k_s, v_s, q_sem, k_sem, v_sem):
    ib = pl.program_id(0)
    i = pl.program_id(1)
    scale = 1.0 / (HEAD_DIM ** 0.5)
    kstart = jnp.maximum(i - 2, 0) * BQ

    q_cp = pltpu.make_async_copy(
        x_ref.at[ib, pl.ds(i * BQ, BQ), pl.ds(0, CH)], q_s, q_sem)
    q_cp.start()

    @pl.when(i == 0)
    def _load_panels():
        pltpu.make_async_copy(
            x_ref.at[ib, :, pl.ds(CH, CH)], k_s, k_sem).start()
        pltpu.make_async_copy(
            x_ref.at[ib, :, pl.ds(2 * CH, CH)], v_s, v_sem).start()
        pltpu.make_async_copy(
            x_ref.at[ib, :, pl.ds(CH, CH)], k_s, k_sem).wait()
        pltpu.make_async_copy(
            x_ref.at[ib, :, pl.ds(2 * CH, CH)], v_s, v_sem).wait()

    q_cp.wait()

    q_idx = i * BQ + jax.lax.broadcasted_iota(jnp.int32, (BQ, KB), 0)
    kv_idx = kstart + jax.lax.broadcasted_iota(jnp.int32, (BQ, KB), 1)
    diff = q_idx - kv_idx
    mask = (diff >= 0) & (diff <= WINDOW)
    bias = jnp.where(mask, jnp.float32(0), jnp.float32(float("-inf")))
    for hh in range(NUM_HEADS):
        c0 = hh * HEAD_DIM
        qh = q_s[:, c0:c0 + HEAD_DIM] * scale
        kh = k_s[pl.ds(kstart, KB), c0:c0 + HEAD_DIM]
        vh = v_s[pl.ds(kstart, KB), c0:c0 + HEAD_DIM]
        s = jax.lax.dot_general(
            qh, kh, (((1,), (1,)), ((), ())),
            preferred_element_type=jnp.float32) + bias
        m = jnp.max(s, axis=-1, keepdims=True)
        p = jnp.exp(s - m)
        denom = jnp.sum(p, axis=-1, keepdims=True)
        oh = jax.lax.dot_general(
            p, vh, (((1,), (0,)), ((), ())),
            preferred_element_type=jnp.float32)
        o_ref[0, :, c0:c0 + HEAD_DIM] = oh * (1.0 / denom)


def kernel(qkv):
    b, l, three, h, e = qkv.shape
    x = qkv.reshape(b, l, three * CH)
    nq = l // BQ

    out = pl.pallas_call(
        _attn_kernel,
        grid=(b, nq),
        in_specs=[pl.BlockSpec(memory_space=pltpu.MemorySpace.HBM)],
        out_specs=pl.BlockSpec((1, BQ, CH), lambda ib, i: (ib, i, 0)),
        out_shape=jax.ShapeDtypeStruct((b, l, CH), jnp.float32),
        scratch_shapes=[
            pltpu.VMEM((BQ, CH), jnp.float32),
            pltpu.VMEM((l, CH), jnp.float32),
            pltpu.VMEM((l, CH), jnp.float32),
            pltpu.SemaphoreType.DMA,
            pltpu.SemaphoreType.DMA,
            pltpu.SemaphoreType.DMA,
        ],
    )(x)

    return out.reshape(b, l, h, e)
